# baseline (speedup 1.0000x reference)
import functools

import jax
import jax.numpy as jnp
from jax import lax
from jax.experimental import pallas as pl
from jax.experimental.pallas import tpu as pltpu
from jax.experimental.pallas import tpu_sc as plsc

H = 128
V = 10000
VP = 10240
E = 320000
NS = 16
GW = 80
NG = E // GW
GPT = NG // NS
IB = 10
NCH = GPT // IB
CHW = IB * GW
RPT = VP // NS
SXC = 64
NSLOT = 3
ZR = NSLOT * GW
DR = VP // H
DRT = DR // NS


def _sc_body(x, idxr, out_pq, out_deg, acc, deg_sh, ibuf, rows, hist, iidx,
             gsem0, gsem1, ssem0, ssem1, isem):
    c = lax.axis_index("c")
    s = lax.axis_index("s")
    gsel = 1 - c
    base = s * RPT
    crow = s * NCH
    gsems = (gsem0, gsem1)
    ssems = (ssem0, ssem1)
    ones16 = jnp.ones((16,), jnp.float32)

    def zrow(i, carry):
        for j in range(H // 16):
            rows[i, pl.ds(j * 16, 16)] = jnp.zeros((16,), jnp.float32)
        return carry

    lax.fori_loop(0, ZR, zrow, 0)

    def hrow(i, carry):
        for j in range(H // 16):
            hist[i, pl.ds(j * 16, 16)] = jnp.zeros((16,), jnp.float32)
        return carry

    lax.fori_loop(0, DR, hrow, 0)
    for k in range(DR // 16):
        iidx[0, pl.ds(k * 16, 16)] = lax.iota(jnp.int32, 16) + (k * 16)

    pltpu.sync_copy(rows, acc.at[pl.ds(base, ZR)])
    pltpu.sync_copy(rows, acc.at[pl.ds(base + ZR, ZR)])
    pltpu.sync_copy(rows.at[pl.ds(0, RPT - 2 * ZR)],
                    acc.at[pl.ds(base + 2 * ZR, RPT - 2 * ZR)])
    pltpu.sync_copy(rows.at[pl.ds(0, DRT)], deg_sh.at[pl.ds(s * DRT, DRT)])
    plsc.subcore_barrier()

    def locate(g):
        ci = g // IB
        return lax.rem(ci, 2), g - ci * IB, lax.rem(g, NSLOT) * GW

    def load_chunk(ci, buf, fire_only):
        for e in range(2):
            d = pltpu.make_async_copy(
                idxr.at[e, pl.ds((crow + ci) * CHW, CHW)], ibuf.at[buf, e],
                isem)
            if fire_only:
                d.start()
            else:
                d.wait()

    def fire_gather(g, parity):
        buf, j, slot = locate(g)
        pltpu.async_copy(x.at[ibuf.at[buf, gsel, pl.ds(j * GW, GW)]],
                         rows.at[pl.ds(slot, GW)], gsems[parity])

    def wait_gather(g, parity):
        buf, j, slot = locate(g)
        pltpu.make_async_copy(x.at[ibuf.at[buf, gsel, pl.ds(j * GW, GW)]],
                              rows.at[pl.ds(slot, GW)], gsems[parity]).wait()

    def fire_scatter(g, parity):
        buf, j, slot = locate(g)
        pltpu.async_copy(rows.at[pl.ds(slot, GW)],
                         acc.at[ibuf.at[buf, c, pl.ds(j * GW, GW)]],
                         ssems[parity], add=True)

    def wait_scatter(g, parity):
        buf, j, slot = locate(g)
        pltpu.make_async_copy(rows.at[pl.ds(slot, GW)],
                              acc.at[ibuf.at[buf, c, pl.ds(j * GW, GW)]],
                              ssems[parity]).wait()

    def hist_update(g):
        buf, j, _ = locate(g)
        for k in range(GW // 16):
            v = ibuf[buf, c, pl.ds(j * GW + k * 16, 16)]
            plsc.addupdate_scatter(
                hist,
                [lax.shift_right_logical(v, 7), jnp.bitwise_and(v, 127)],
                ones16)

    load_chunk(0, 0, fire_only=True)
    load_chunk(0, 0, fire_only=False)
    fire_gather(0, 0)
    fire_gather(1, 1)

    def step(b, carry):
        for p in range(2):
            g = 2 * b + p
            wait_gather(g, p)
            fire_scatter(g, p)
            hist_update(g)

            @pl.when(g >= 1)
            def _():
                wait_scatter(g - 1, 1 - p)

            ci = g // IB
            j = g - ci * IB

            @pl.when(jnp.logical_and(j == 0, ci + 1 < NCH))
            def _():
                load_chunk(ci + 1, 1 - lax.rem(ci, 2), fire_only=True)

            @pl.when(g + 2 < GPT)
            def _():
                g2 = g + 2
                ci2 = g2 // IB

                @pl.when(g2 - ci2 * IB == 0)
                def _():
                    load_chunk(ci2, lax.rem(ci2, 2), fire_only=False)

                fire_gather(g2, p)
        return carry

    lax.fori_loop(0, GPT // 2, step, 0)
    wait_scatter(GPT - 1, (GPT - 1) % 2)
    pltpu.sync_copy(hist, deg_sh.at[iidx.at[0]], add=True)
    plsc.subcore_barrier()

    pq_desc = pltpu.make_async_copy(
        acc.at[pl.ds(base, RPT)], out_pq.at[c, pl.ds(base, RPT)], isem)
    pq_desc.start()
    pltpu.sync_copy(
        deg_sh.at[pl.ds(s * DRT, DRT)],
        out_deg.at[c, s // 2, pl.ds(lax.rem(s, 2) * DRT, DRT)])
    pq_desc.wait()


_sc_accumulate = functools.partial(
    pl.kernel,
    out_type=(
        jax.ShapeDtypeStruct((2, VP, H), jnp.float32),
        jax.ShapeDtypeStruct((2, DR // 10, 10, H), jnp.float32),
    ),
    mesh=plsc.VectorSubcoreMesh(core_axis_name="c", subcore_axis_name="s"),
    compiler_params=pltpu.CompilerParams(use_tc_tiling_on_sc=False,
                                         needs_layout_passes=False),
    scratch_types=[
        pltpu.VMEM_SHARED((VP, H), jnp.float32),
        pltpu.VMEM_SHARED((DR, H), jnp.float32),
        pltpu.VMEM((2, 2, CHW), jnp.int32),
        pltpu.VMEM((ZR, H), jnp.float32),
        pltpu.VMEM((DR, H), jnp.float32),
        pltpu.VMEM((1, DR), jnp.int32),
        pltpu.SemaphoreType.DMA,
        pltpu.SemaphoreType.DMA,
        pltpu.SemaphoreType.DMA,
        pltpu.SemaphoreType.DMA,
        pltpu.SemaphoreType.DMA,
    ],
)(_sc_body)


def _mm_body(d_ref, x_ref, p_ref, q_ref, w_ref, o_ref):
    xb = x_ref[...]
    sel = jnp.where(
        lax.broadcasted_iota(jnp.int32, (_BM, _DB), 0) // H
        == lax.broadcasted_iota(jnp.int32, (_BM, _DB), 1),
        1.0, 0.0)
    lane_pick = jnp.where(
        lax.broadcasted_iota(jnp.int32, (_BM, H), 1)
        == lax.rem(lax.broadcasted_iota(jnp.int32, (_BM, H), 0), H),
        1.0, 0.0)

    def spread(u):
        z = lax.dot_general(sel, u, (((1,), (0,)), ((), ())),
                            preferred_element_type=jnp.float32)
        return jnp.sum(z * lane_pick, axis=1, keepdims=True)

    dout = spread(d_ref[0, 0])
    din = spread(d_ref[1, 0])
    hcat = jnp.concatenate(
        [xb * dout, p_ref[0], q_ref[0], xb * din], axis=1)
    w = w_ref[...]
    wcat = jnp.concatenate([w[:H, :], w[H:, :]], axis=1)
    o_ref[...] = lax.dot_general(
        hcat, wcat, (((1,), (1,)), ((), ())),
        preferred_element_type=jnp.float32)


_BM = 1280
_DB = _BM // H
_mm = pl.pallas_call(
    _mm_body,
    grid=((V + _BM - 1) // _BM,),
    in_specs=[
        pl.BlockSpec((2, 1, _DB, H), lambda i: (0, i, 0, 0)),
        pl.BlockSpec((_BM, H), lambda i: (i, 0)),
        pl.BlockSpec((1, _BM, H), lambda i: (0, i, 0)),
        pl.BlockSpec((1, _BM, H), lambda i: (1, i, 0)),
        pl.BlockSpec((2 * H, 2 * H), lambda i: (0, 0)),
    ],
    out_specs=pl.BlockSpec((_BM, H), lambda i: (i, 0)),
    out_shape=jax.ShapeDtypeStruct((V, H), jnp.float32),
)


def kernel(x, edge_index, W):
    pq, deg4 = _sc_accumulate(x, edge_index)
    return _mm(deg4, x, pq, pq, W)


# speedup vs baseline: 1.0035x; 1.0035x over previous
import functools

import jax
import jax.numpy as jnp
from jax import lax
from jax.experimental import pallas as pl
from jax.experimental.pallas import tpu as pltpu
from jax.experimental.pallas import tpu_sc as plsc

H = 128
V = 10000
VP = 10240
E = 320000
NS = 16
GW = 80
NG = E // GW
GPT = NG // NS
IB = 10
NCH = GPT // IB
CHW = IB * GW
RPT = VP // NS
NSLOT = 3
ZR = NSLOT * GW
DR = VP // H
DRT = DR // NS


def _sc_body(x, idxr, out_pq, out_deg, acc, deg_sh, ibuf, rows, hist, iidx,
             gsem0, gsem1, ssem0, ssem1, isem):
    c = lax.axis_index("c")
    s = lax.axis_index("s")
    gsel = 1 - c
    base = s * RPT
    crow = s * NCH
    gsems = (gsem0, gsem1)
    ssems = (ssem0, ssem1)
    ones16 = jnp.ones((16,), jnp.float32)

    def zrow(i, carry):
        for j in range(H // 16):
            rows[i, pl.ds(j * 16, 16)] = jnp.zeros((16,), jnp.float32)
        return carry

    lax.fori_loop(0, ZR, zrow, 0)

    def hrow(i, carry):
        for j in range(H // 16):
            hist[i, pl.ds(j * 16, 16)] = jnp.zeros((16,), jnp.float32)
        return carry

    lax.fori_loop(0, DR, hrow, 0)
    for k in range(DR // 16):
        iidx[0, pl.ds(k * 16, 16)] = lax.iota(jnp.int32, 16) + (k * 16)

    pltpu.sync_copy(rows, acc.at[pl.ds(base, ZR)])
    pltpu.sync_copy(rows, acc.at[pl.ds(base + ZR, ZR)])
    pltpu.sync_copy(rows.at[pl.ds(0, RPT - 2 * ZR)],
                    acc.at[pl.ds(base + 2 * ZR, RPT - 2 * ZR)])
    pltpu.sync_copy(rows.at[pl.ds(0, DRT)], deg_sh.at[pl.ds(s * DRT, DRT)])
    plsc.subcore_barrier()

    def locate(g):
        ci = g // IB
        return lax.rem(ci, 2), g - ci * IB, lax.rem(g, NSLOT) * GW

    def load_chunk(ci, buf, fire_only):
        for e in range(2):
            d = pltpu.make_async_copy(
                idxr.at[e, pl.ds((crow + ci) * CHW, CHW)], ibuf.at[buf, e],
                isem)
            if fire_only:
                d.start()
            else:
                d.wait()

    def fire_gather(g, parity):
        buf, j, slot = locate(g)
        pltpu.async_copy(x.at[ibuf.at[buf, gsel, pl.ds(j * GW, GW)]],
                         rows.at[pl.ds(slot, GW)], gsems[parity])

    def wait_gather(g, parity):
        buf, j, slot = locate(g)
        pltpu.make_async_copy(x.at[ibuf.at[buf, gsel, pl.ds(j * GW, GW)]],
                              rows.at[pl.ds(slot, GW)], gsems[parity]).wait()

    def fire_scatter(g, parity):
        buf, j, slot = locate(g)
        pltpu.async_copy(rows.at[pl.ds(slot, GW)],
                         acc.at[ibuf.at[buf, c, pl.ds(j * GW, GW)]],
                         ssems[parity], add=True)

    def wait_scatter(g, parity):
        buf, j, slot = locate(g)
        pltpu.make_async_copy(rows.at[pl.ds(slot, GW)],
                              acc.at[ibuf.at[buf, c, pl.ds(j * GW, GW)]],
                              ssems[parity]).wait()

    def hist_update(g):
        buf, j, _ = locate(g)
        for k in range(GW // 16):
            v = ibuf[buf, c, pl.ds(j * GW + k * 16, 16)]
            plsc.addupdate_scatter(
                hist,
                [lax.shift_right_logical(v, 7), jnp.bitwise_and(v, 127)],
                ones16)

    load_chunk(0, 0, fire_only=True)
    load_chunk(0, 0, fire_only=False)
    fire_gather(0, 0)
    fire_gather(1, 1)

    def step(b, carry):
        for p in range(2):
            g = 2 * b + p
            wait_gather(g, p)
            fire_scatter(g, p)
            hist_update(g)

            @pl.when(g >= 1)
            def _():
                wait_scatter(g - 1, 1 - p)

            ci = g // IB
            j = g - ci * IB

            @pl.when(jnp.logical_and(j == 0, ci + 1 < NCH))
            def _():
                load_chunk(ci + 1, 1 - lax.rem(ci, 2), fire_only=True)

            @pl.when(g + 2 < GPT)
            def _():
                g2 = g + 2
                ci2 = g2 // IB

                @pl.when(g2 - ci2 * IB == 0)
                def _():
                    load_chunk(ci2, lax.rem(ci2, 2), fire_only=False)

                fire_gather(g2, p)
        return carry

    lax.fori_loop(0, GPT // 2, step, 0)
    wait_scatter(GPT - 1, (GPT - 1) % 2)
    pltpu.sync_copy(hist, deg_sh.at[iidx.at[0]], add=True)
    plsc.subcore_barrier()

    pq_desc = pltpu.make_async_copy(
        acc.at[pl.ds(base, RPT)], out_pq.at[c, pl.ds(base, RPT)], isem)
    pq_desc.start()
    pltpu.sync_copy(
        deg_sh.at[pl.ds(s * DRT, DRT)],
        out_deg.at[c, s // 2, pl.ds(lax.rem(s, 2) * DRT, DRT)])
    pq_desc.wait()


_sc_accumulate = functools.partial(
    pl.kernel,
    out_type=(
        jax.ShapeDtypeStruct((2, VP, H), jnp.float32),
        jax.ShapeDtypeStruct((2, DR // 10, 10, H), jnp.float32),
    ),
    mesh=plsc.VectorSubcoreMesh(core_axis_name="c", subcore_axis_name="s"),
    compiler_params=pltpu.CompilerParams(use_tc_tiling_on_sc=False,
                                         needs_layout_passes=False),
    scratch_types=[
        pltpu.VMEM_SHARED((VP, H), jnp.float32),
        pltpu.VMEM_SHARED((DR, H), jnp.float32),
        pltpu.VMEM((2, 2, CHW), jnp.int32),
        pltpu.VMEM((ZR, H), jnp.float32),
        pltpu.VMEM((DR, H), jnp.float32),
        pltpu.VMEM((1, DR), jnp.int32),
        pltpu.SemaphoreType.DMA,
        pltpu.SemaphoreType.DMA,
        pltpu.SemaphoreType.DMA,
        pltpu.SemaphoreType.DMA,
        pltpu.SemaphoreType.DMA,
    ],
)(_sc_body)


def _mm_body(d_ref, x_ref, p_ref, q_ref, w_ref, o_ref):
    xb = x_ref[...]
    sel = jnp.where(
        lax.broadcasted_iota(jnp.int32, (_BM, _DB), 0) // H
        == lax.broadcasted_iota(jnp.int32, (_BM, _DB), 1),
        1.0, 0.0)
    lane_pick = jnp.where(
        lax.broadcasted_iota(jnp.int32, (_BM, H), 1)
        == lax.rem(lax.broadcasted_iota(jnp.int32, (_BM, H), 0), H),
        1.0, 0.0)

    def spread(u):
        z = lax.dot_general(sel, u, (((1,), (0,)), ((), ())),
                            preferred_element_type=jnp.float32)
        return jnp.sum(z * lane_pick, axis=1, keepdims=True)

    dout = spread(d_ref[0, 0])
    din = spread(d_ref[1, 0])
    hcat = jnp.concatenate(
        [xb * dout, p_ref[0], q_ref[0], xb * din], axis=1)
    w = w_ref[...]
    wcat = jnp.concatenate([w[:H, :], w[H:, :]], axis=1)
    o_ref[...] = lax.dot_general(
        hcat, wcat, (((1,), (1,)), ((), ())),
        preferred_element_type=jnp.float32)


_BM = 1280
_DB = _BM // H
_mm = pl.pallas_call(
    _mm_body,
    grid=((V + _BM - 1) // _BM,),
    in_specs=[
        pl.BlockSpec((2, 1, _DB, H), lambda i: (0, i, 0, 0)),
        pl.BlockSpec((_BM, H), lambda i: (i, 0)),
        pl.BlockSpec((1, _BM, H), lambda i: (0, i, 0)),
        pl.BlockSpec((1, _BM, H), lambda i: (1, i, 0)),
        pl.BlockSpec((2 * H, 2 * H), lambda i: (0, 0)),
    ],
    out_specs=pl.BlockSpec((_BM, H), lambda i: (i, 0)),
    out_shape=jax.ShapeDtypeStruct((V, H), jnp.float32),
)


def kernel(x, edge_index, W):
    pq, deg4 = _sc_accumulate(x, edge_index)
    return _mm(deg4, x, pq, pq, W)
